# Initial kernel scaffold; baseline (speedup 1.0000x reference)
import jax
import jax.numpy as jnp
from jax.experimental import pallas as pl


def kernel(x, edge_index, emb_table, conv_W, lin_W):
    raise NotImplementedError("write your pallas kernel here")


# speedup vs baseline: 4.8719x; 4.8719x over previous
import functools

import jax
import jax.numpy as jnp
from jax import lax
from jax.experimental import pallas as pl
from jax.experimental.pallas import tpu as pltpu
from jax.experimental.pallas import tpu_sc as plsc

N = 10000
H = 128
NP = 10240
EP = 327680
NW = 32
TILES = 16
ROWS_PER_TILE = NP // TILES
CH = 128
J = 2
BM = 1024


def _sc_segsum(width, edges_per_tile, gather):
  n_iter = edges_per_tile // (CH * J)
  rows_per_iter = edges_per_tile // CH
  mesh = plsc.VectorSubcoreMesh(core_axis_name="c", subcore_axis_name="s")

  def body(table, src2, dst2, z_op, fill_op, out, srcb, dstb, rows, acc,
           gsem, ssem):
    c = lax.axis_index("c")
    s = lax.axis_index("s")
    w = c * TILES + s
    pltpu.sync_copy(z_op, rows.at[pl.ds(0, CH)])
    r0 = s * ROWS_PER_TILE
    for k in range(ROWS_PER_TILE // CH):
      pltpu.sync_copy(rows.at[pl.ds(0, CH)], acc.at[pl.ds(r0 + k * CH, CH)])
    if not gather:
      pltpu.sync_copy(fill_op, rows)
    plsc.subcore_barrier()

    tile_row0 = w * rows_per_iter

    @pl.loop(0, n_iter)
    def _(i):
      row0 = tile_row0 + i * J
      pltpu.sync_copy(dst2.at[pl.ds(row0, J)], dstb)
      if gather:
        pltpu.sync_copy(src2.at[pl.ds(row0, J)], srcb)
        gs = [
            pltpu.async_copy(table.at[srcb.at[j]],
                             rows.at[pl.ds(j * CH, CH)], gsem)
            for j in range(J)
        ]
        for g in gs:
          g.wait()
      ss = [
          pltpu.async_copy(rows.at[pl.ds(j * CH, CH)],
                           acc.at[dstb.at[j]], ssem, add=True)
          for j in range(J)
      ]
      for sc in ss:
        sc.wait()

    plsc.subcore_barrier()
    pltpu.sync_copy(acc.at[pl.ds(r0, ROWS_PER_TILE)],
                    out.at[c, pl.ds(r0, ROWS_PER_TILE)])

  return functools.partial(
      pl.kernel,
      out_type=jax.ShapeDtypeStruct((2, NP, width), jnp.float32),
      mesh=mesh,
      scratch_types=[
          pltpu.VMEM((J, CH), jnp.int32),
          pltpu.VMEM((J, CH), jnp.int32),
          pltpu.VMEM((J * CH, width), jnp.float32),
          pltpu.VMEM_SHARED((NP, width), jnp.float32),
          pltpu.SemaphoreType.DMA,
          pltpu.SemaphoreType.DMA,
      ],
  )(body)


def _tc_first(h0a, h0b, deg0, deg1, W0):
  def body(a_ref, b_ref, d0_ref, d1_ref, w_ref, hws_ref, dis_ref):
    dis = lax.rsqrt(d0_ref[...] + d1_ref[...] + 1.0)
    h = jnp.maximum((a_ref[...] + b_ref[...]) * (1.0 / 16.0), 0.0)
    hw = jnp.dot(h, w_ref[...], preferred_element_type=jnp.float32)
    hws_ref[...] = hw * dis
    dis_ref[...] = dis

  return pl.pallas_call(
      body,
      grid=(NP // BM,),
      in_specs=[
          pl.BlockSpec((BM, H), lambda m: (m, 0)),
          pl.BlockSpec((BM, H), lambda m: (m, 0)),
          pl.BlockSpec((BM, 1), lambda m: (m, 0)),
          pl.BlockSpec((BM, 1), lambda m: (m, 0)),
          pl.BlockSpec((H, H), lambda m: (0, 0)),
      ],
      out_specs=[
          pl.BlockSpec((BM, H), lambda m: (m, 0)),
          pl.BlockSpec((BM, 1), lambda m: (m, 0)),
      ],
      out_shape=[
          jax.ShapeDtypeStruct((NP, H), jnp.float32),
          jax.ShapeDtypeStruct((NP, 1), jnp.float32),
      ],
  )(h0a, h0b, deg0, deg1, W0)


def _tc_mid(s0, s1, hp, dis, W):
  def body(s0_ref, s1_ref, hp_ref, dis_ref, w_ref, out_ref):
    dis = dis_ref[...]
    t = jnp.maximum((s0_ref[...] + s1_ref[...] + hp_ref[...]) * dis, 0.0)
    out_ref[...] = jnp.dot(
        t, w_ref[...], preferred_element_type=jnp.float32) * dis

  return pl.pallas_call(
      body,
      grid=(NP // BM,),
      in_specs=[
          pl.BlockSpec((BM, H), lambda m: (m, 0)),
          pl.BlockSpec((BM, H), lambda m: (m, 0)),
          pl.BlockSpec((BM, H), lambda m: (m, 0)),
          pl.BlockSpec((BM, 1), lambda m: (m, 0)),
          pl.BlockSpec((H, H), lambda m: (0, 0)),
      ],
      out_specs=pl.BlockSpec((BM, H), lambda m: (m, 0)),
      out_shape=jax.ShapeDtypeStruct((NP, H), jnp.float32),
  )(s0, s1, hp, dis, W)


def _tc_last(s0, s1, hp, dis, linp):
  def body(s0_ref, s1_ref, hp_ref, dis_ref, w_ref, out_ref):
    t = jnp.maximum(
        (s0_ref[...] + s1_ref[...] + hp_ref[...]) * dis_ref[...], 0.0)
    z = jnp.dot(t, w_ref[...], preferred_element_type=jnp.float32)
    col = lax.broadcasted_iota(jnp.int32, (BM, H), 1)
    z = jnp.where(col < 7, z, -jnp.inf)
    m = jnp.max(z, axis=1, keepdims=True)
    e = jnp.exp(z - m)
    out_ref[...] = e / jnp.sum(e, axis=1, keepdims=True)

  return pl.pallas_call(
      body,
      grid=(NP // BM,),
      in_specs=[
          pl.BlockSpec((BM, H), lambda m: (m, 0)),
          pl.BlockSpec((BM, H), lambda m: (m, 0)),
          pl.BlockSpec((BM, H), lambda m: (m, 0)),
          pl.BlockSpec((BM, 1), lambda m: (m, 0)),
          pl.BlockSpec((H, H), lambda m: (0, 0)),
      ],
      out_specs=pl.BlockSpec((BM, H), lambda m: (m, 0)),
      out_shape=jax.ShapeDtypeStruct((NP, H), jnp.float32),
  )(s0, s1, hp, dis, linp)


def kernel(x, edge_index, emb_table, conv_W, lin_W):
  f32 = jnp.float32
  i32 = jnp.int32
  bag = x.shape[1]

  xp = jnp.pad(x.astype(i32), ((0, NP - N), (0, 0)))
  xflat2 = xp.reshape(NP * bag // CH, CH)
  bag_dst2 = (jnp.arange(NP * bag, dtype=i32) // bag).reshape(
      NP * bag // CH, CH)
  pad_e = EP - edge_index.shape[1]
  srcp = jnp.concatenate(
      [edge_index[0].astype(i32), jnp.zeros((pad_e,), i32)])
  dstp = jnp.concatenate(
      [edge_index[1].astype(i32), jnp.full((pad_e,), NP - 1, i32)])
  src2 = srcp.reshape(EP // CH, CH)
  dst2 = dstp.reshape(EP // CH, CH)
  zeros_h = jnp.zeros((CH, H), f32)
  ones_fill = jnp.ones((J * CH, H), f32)
  ones_tab = jnp.ones((8, H), f32)
  linp = jnp.pad(lin_W.astype(f32), ((0, 0), (0, H - lin_W.shape[1])))

  deg = _sc_segsum(H, EP // NW, gather=False)(
      ones_tab, src2, dst2, zeros_h, ones_fill)
  h0 = _sc_segsum(H, NP * bag // NW, gather=True)(
      emb_table, xflat2, bag_dst2, zeros_h, zeros_h)

  hws, dis = _tc_first(h0[0], h0[1], deg[0, :, :1], deg[1, :, :1], conv_W[0])

  seg = _sc_segsum(H, EP // NW, gather=True)
  for i in range(1, 5):
    scat = seg(hws, src2, dst2, zeros_h, zeros_h)
    hws = _tc_mid(scat[0], scat[1], hws, dis, conv_W[i])
  scat = seg(hws, src2, dst2, zeros_h, zeros_h)
  probs = _tc_last(scat[0], scat[1], hws, dis, linp)
  return probs[:N, :lin_W.shape[1]]


# speedup vs baseline: 5.1170x; 1.0503x over previous
import functools

import jax
import jax.numpy as jnp
from jax import lax
from jax.experimental import pallas as pl
from jax.experimental.pallas import tpu as pltpu
from jax.experimental.pallas import tpu_sc as plsc

N = 10000
H = 128
NP = 10240
EP = 327680
NW = 32
TILES = 16
ROWS_PER_TILE = NP // TILES
CH = 128
J = 2
BM = 1024


def _sc_segsum(width, edges_per_tile, gather):
  n = edges_per_tile // CH
  mesh = plsc.VectorSubcoreMesh(core_axis_name="c", subcore_axis_name="s")

  def body(table, src2, dst2, z_op, fill_op, out, srcb, dstb, rows, acc,
           gsem, ssem):
    c = lax.axis_index("c")
    s = lax.axis_index("s")
    w = c * TILES + s
    r0 = s * ROWS_PER_TILE
    pltpu.sync_copy(z_op, rows.at[0])
    for k in range(ROWS_PER_TILE // CH):
      pltpu.sync_copy(rows.at[0], acc.at[pl.ds(r0 + k * CH, CH)])
    if not gather:
      pltpu.sync_copy(fill_op, rows.at[0])
      pltpu.sync_copy(fill_op, rows.at[1])
    plsc.subcore_barrier()

    tile_row0 = w * n

    pltpu.sync_copy(dst2.at[tile_row0], dstb.at[0])
    if gather:
      pltpu.sync_copy(src2.at[tile_row0], srcb.at[0])
      pltpu.async_copy(table.at[srcb.at[0]], rows.at[0], gsem)

    @pl.loop(0, n // 2)
    def _(k):
      for b in range(2):
        t = 2 * k + b
        p, q = b, 1 - b
        if gather:
          pltpu.make_async_copy(table.at[srcb.at[p]], rows.at[p],
                                gsem).wait()
        pltpu.async_copy(rows.at[p], acc.at[dstb.at[p]], ssem, add=True)

        @pl.when(t > 0)
        def _():
          pltpu.make_async_copy(rows.at[q], acc.at[dstb.at[q]], ssem).wait()

        @pl.when(t < n - 1)
        def _():
          pltpu.sync_copy(dst2.at[t + 1 + tile_row0], dstb.at[q])
          if gather:
            pltpu.sync_copy(src2.at[t + 1 + tile_row0], srcb.at[q])
            pltpu.async_copy(table.at[srcb.at[q]], rows.at[q], gsem)

    pltpu.make_async_copy(rows.at[1], acc.at[dstb.at[1]], ssem).wait()
    plsc.subcore_barrier()
    pltpu.sync_copy(acc.at[pl.ds(r0, ROWS_PER_TILE)],
                    out.at[c, pl.ds(r0, ROWS_PER_TILE)])

  return functools.partial(
      pl.kernel,
      out_type=jax.ShapeDtypeStruct((2, NP, width), jnp.float32),
      mesh=mesh,
      scratch_types=[
          pltpu.VMEM((2, CH), jnp.int32),
          pltpu.VMEM((2, CH), jnp.int32),
          pltpu.VMEM((2, CH, width), jnp.float32),
          pltpu.VMEM_SHARED((NP, width), jnp.float32),
          pltpu.SemaphoreType.DMA,
          pltpu.SemaphoreType.DMA,
      ],
  )(body)


def _tc_first(h0a, h0b, deg0, deg1, W0):
  def body(a_ref, b_ref, d0_ref, d1_ref, w_ref, hws_ref, dis_ref):
    dis = lax.rsqrt(d0_ref[...] + d1_ref[...] + 1.0)
    h = jnp.maximum((a_ref[...] + b_ref[...]) * (1.0 / 16.0), 0.0)
    hw = jnp.dot(h, w_ref[...], preferred_element_type=jnp.float32)
    hws_ref[...] = hw * dis
    dis_ref[...] = dis

  return pl.pallas_call(
      body,
      grid=(NP // BM,),
      in_specs=[
          pl.BlockSpec((BM, H), lambda m: (m, 0)),
          pl.BlockSpec((BM, H), lambda m: (m, 0)),
          pl.BlockSpec((BM, 1), lambda m: (m, 0)),
          pl.BlockSpec((BM, 1), lambda m: (m, 0)),
          pl.BlockSpec((H, H), lambda m: (0, 0)),
      ],
      out_specs=[
          pl.BlockSpec((BM, H), lambda m: (m, 0)),
          pl.BlockSpec((BM, 1), lambda m: (m, 0)),
      ],
      out_shape=[
          jax.ShapeDtypeStruct((NP, H), jnp.float32),
          jax.ShapeDtypeStruct((NP, 1), jnp.float32),
      ],
  )(h0a, h0b, deg0, deg1, W0)


def _tc_mid(s0, s1, hp, dis, W):
  def body(s0_ref, s1_ref, hp_ref, dis_ref, w_ref, out_ref):
    dis = dis_ref[...]
    t = jnp.maximum((s0_ref[...] + s1_ref[...] + hp_ref[...]) * dis, 0.0)
    out_ref[...] = jnp.dot(
        t, w_ref[...], preferred_element_type=jnp.float32) * dis

  return pl.pallas_call(
      body,
      grid=(NP // BM,),
      in_specs=[
          pl.BlockSpec((BM, H), lambda m: (m, 0)),
          pl.BlockSpec((BM, H), lambda m: (m, 0)),
          pl.BlockSpec((BM, H), lambda m: (m, 0)),
          pl.BlockSpec((BM, 1), lambda m: (m, 0)),
          pl.BlockSpec((H, H), lambda m: (0, 0)),
      ],
      out_specs=pl.BlockSpec((BM, H), lambda m: (m, 0)),
      out_shape=jax.ShapeDtypeStruct((NP, H), jnp.float32),
  )(s0, s1, hp, dis, W)


def _tc_last(s0, s1, hp, dis, linp):
  def body(s0_ref, s1_ref, hp_ref, dis_ref, w_ref, out_ref):
    t = jnp.maximum(
        (s0_ref[...] + s1_ref[...] + hp_ref[...]) * dis_ref[...], 0.0)
    z = jnp.dot(t, w_ref[...], preferred_element_type=jnp.float32)
    col = lax.broadcasted_iota(jnp.int32, (BM, H), 1)
    z = jnp.where(col < 7, z, -jnp.inf)
    m = jnp.max(z, axis=1, keepdims=True)
    e = jnp.exp(z - m)
    out_ref[...] = e / jnp.sum(e, axis=1, keepdims=True)

  return pl.pallas_call(
      body,
      grid=(NP // BM,),
      in_specs=[
          pl.BlockSpec((BM, H), lambda m: (m, 0)),
          pl.BlockSpec((BM, H), lambda m: (m, 0)),
          pl.BlockSpec((BM, H), lambda m: (m, 0)),
          pl.BlockSpec((BM, 1), lambda m: (m, 0)),
          pl.BlockSpec((H, H), lambda m: (0, 0)),
      ],
      out_specs=pl.BlockSpec((BM, H), lambda m: (m, 0)),
      out_shape=jax.ShapeDtypeStruct((NP, H), jnp.float32),
  )(s0, s1, hp, dis, linp)


def kernel(x, edge_index, emb_table, conv_W, lin_W):
  f32 = jnp.float32
  i32 = jnp.int32
  bag = x.shape[1]

  xp = jnp.pad(x.astype(i32), ((0, NP - N), (0, 0)))
  xflat2 = xp.reshape(NP * bag // CH, CH)
  bag_dst2 = (jnp.arange(NP * bag, dtype=i32) // bag).reshape(
      NP * bag // CH, CH)
  pad_e = EP - edge_index.shape[1]
  srcp = jnp.concatenate(
      [edge_index[0].astype(i32), jnp.zeros((pad_e,), i32)])
  dstp = jnp.concatenate(
      [edge_index[1].astype(i32), jnp.full((pad_e,), NP - 1, i32)])
  src2 = srcp.reshape(EP // CH, CH)
  dst2 = dstp.reshape(EP // CH, CH)
  zeros_h = jnp.zeros((CH, H), f32)
  ones_fill = jnp.ones((CH, H), f32)
  ones_tab = jnp.ones((8, H), f32)
  linp = jnp.pad(lin_W.astype(f32), ((0, 0), (0, H - lin_W.shape[1])))

  deg = _sc_segsum(H, EP // NW, gather=False)(
      ones_tab, src2, dst2, zeros_h, ones_fill)
  h0 = _sc_segsum(H, NP * bag // NW, gather=True)(
      emb_table, xflat2, bag_dst2, zeros_h, zeros_h)

  hws, dis = _tc_first(h0[0], h0[1], deg[0, :, :1], deg[1, :, :1], conv_W[0])

  seg = _sc_segsum(H, EP // NW, gather=True)
  for i in range(1, 5):
    scat = seg(hws, src2, dst2, zeros_h, zeros_h)
    hws = _tc_mid(scat[0], scat[1], hws, dis, conv_W[i])
  scat = seg(hws, src2, dst2, zeros_h, zeros_h)
  probs = _tc_last(scat[0], scat[1], hws, dis, linp)
  return probs[:N, :lin_W.shape[1]]


# speedup vs baseline: 10.5694x; 2.0655x over previous
import functools

import jax
import jax.numpy as jnp
from jax import lax
from jax.experimental import pallas as pl
from jax.experimental.pallas import tpu as pltpu
from jax.experimental.pallas import tpu_sc as plsc

N = 10000
H = 128
NP = 10240
EP = 327680
NW = 32
TILES = 16
ROWS_PER_TILE = NP // TILES
CH = 128
J = 2
BM = 1024


def _sc_segsum(width, edges_per_tile, gather):
  n = edges_per_tile // CH
  mesh = plsc.VectorSubcoreMesh(core_axis_name="c", subcore_axis_name="s")

  def body(table, src2, dst2, z_op, fill_op, out, srcb, dstb, rows, acc,
           gsem, ssem):
    c = lax.axis_index("c")
    s = lax.axis_index("s")
    w = c * TILES + s
    r0 = s * ROWS_PER_TILE
    pltpu.sync_copy(z_op, rows.at[0])
    for k in range(ROWS_PER_TILE // CH):
      pltpu.sync_copy(rows.at[0], acc.at[pl.ds(r0 + k * CH, CH)])
    if not gather:
      pltpu.sync_copy(fill_op, rows.at[0])
      pltpu.sync_copy(fill_op, rows.at[1])
    plsc.subcore_barrier()

    tile_row0 = w * n

    pltpu.sync_copy(dst2.at[tile_row0], dstb.at[0])
    if gather:
      pltpu.sync_copy(src2.at[tile_row0], srcb.at[0])
      pltpu.async_copy(table.at[srcb.at[0]], rows.at[0], gsem)

    @pl.loop(0, n // 2)
    def _(k):
      for b in range(2):
        t = 2 * k + b
        p, q = b, 1 - b
        if gather:
          pltpu.make_async_copy(table.at[srcb.at[p]], rows.at[p],
                                gsem).wait()
        pltpu.async_copy(rows.at[p], acc.at[dstb.at[p]], ssem, add=True)

        @pl.when(t > 0)
        def _():
          pltpu.make_async_copy(rows.at[q], acc.at[dstb.at[q]], ssem).wait()

        @pl.when(t < n - 1)
        def _():
          pltpu.sync_copy(dst2.at[t + 1 + tile_row0], dstb.at[q])
          if gather:
            pltpu.sync_copy(src2.at[t + 1 + tile_row0], srcb.at[q])
            pltpu.async_copy(table.at[srcb.at[q]], rows.at[q], gsem)

    pltpu.make_async_copy(rows.at[1], acc.at[dstb.at[1]], ssem).wait()
    plsc.subcore_barrier()
    pltpu.sync_copy(acc.at[pl.ds(r0, ROWS_PER_TILE)],
                    out.at[c, pl.ds(r0, ROWS_PER_TILE)])

  return functools.partial(
      pl.kernel,
      out_type=jax.ShapeDtypeStruct((2, NP, width), jnp.float32),
      mesh=mesh,
      scratch_types=[
          pltpu.VMEM((2, CH), jnp.int32),
          pltpu.VMEM((2, CH), jnp.int32),
          pltpu.VMEM((2, CH, width), jnp.float32),
          pltpu.VMEM_SHARED((NP, width), jnp.float32),
          pltpu.SemaphoreType.DMA,
          pltpu.SemaphoreType.DMA,
      ],
  )(body)


def _tc_first(h0a, h0b, deg0, deg1, W0):
  def body(a_ref, b_ref, d0_ref, d1_ref, w_ref, hws_ref, dis_ref):
    dis = lax.rsqrt(d0_ref[...] + d1_ref[...] + 1.0)
    h = jnp.maximum((a_ref[...] + b_ref[...]) * (1.0 / 16.0), 0.0)
    hw = jnp.dot(h, w_ref[...], preferred_element_type=jnp.float32)
    hws_ref[...] = hw * dis
    dis_ref[...] = dis

  return pl.pallas_call(
      body,
      grid=(NP // BM,),
      in_specs=[
          pl.BlockSpec((BM, H), lambda m: (m, 0)),
          pl.BlockSpec((BM, H), lambda m: (m, 0)),
          pl.BlockSpec((BM, 1), lambda m: (m, 0)),
          pl.BlockSpec((BM, 1), lambda m: (m, 0)),
          pl.BlockSpec((H, H), lambda m: (0, 0)),
      ],
      out_specs=[
          pl.BlockSpec((BM, H), lambda m: (m, 0)),
          pl.BlockSpec((BM, 1), lambda m: (m, 0)),
      ],
      out_shape=[
          jax.ShapeDtypeStruct((NP, H), jnp.float32),
          jax.ShapeDtypeStruct((NP, 1), jnp.float32),
      ],
  )(h0a, h0b, deg0, deg1, W0)


def _tc_mid(s0, s1, hp, dis, W):
  def body(s0_ref, s1_ref, hp_ref, dis_ref, w_ref, out_ref):
    dis = dis_ref[...]
    t = jnp.maximum((s0_ref[...] + s1_ref[...] + hp_ref[...]) * dis, 0.0)
    out_ref[...] = jnp.dot(
        t, w_ref[...], preferred_element_type=jnp.float32) * dis

  return pl.pallas_call(
      body,
      grid=(NP // BM,),
      in_specs=[
          pl.BlockSpec((BM, H), lambda m: (m, 0)),
          pl.BlockSpec((BM, H), lambda m: (m, 0)),
          pl.BlockSpec((BM, H), lambda m: (m, 0)),
          pl.BlockSpec((BM, 1), lambda m: (m, 0)),
          pl.BlockSpec((H, H), lambda m: (0, 0)),
      ],
      out_specs=pl.BlockSpec((BM, H), lambda m: (m, 0)),
      out_shape=jax.ShapeDtypeStruct((NP, H), jnp.float32),
  )(s0, s1, hp, dis, W)


def _tc_last(s0, s1, hp, dis, linp):
  def body(s0_ref, s1_ref, hp_ref, dis_ref, w_ref, out_ref):
    t = jnp.maximum(
        (s0_ref[...] + s1_ref[...] + hp_ref[...]) * dis_ref[...], 0.0)
    z = jnp.dot(t, w_ref[...], preferred_element_type=jnp.float32)
    col = lax.broadcasted_iota(jnp.int32, (BM, H), 1)
    z = jnp.where(col < 7, z, -jnp.inf)
    m = jnp.max(z, axis=1, keepdims=True)
    e = jnp.exp(z - m)
    out_ref[...] = e / jnp.sum(e, axis=1, keepdims=True)

  return pl.pallas_call(
      body,
      grid=(NP // BM,),
      in_specs=[
          pl.BlockSpec((BM, H), lambda m: (m, 0)),
          pl.BlockSpec((BM, H), lambda m: (m, 0)),
          pl.BlockSpec((BM, H), lambda m: (m, 0)),
          pl.BlockSpec((BM, 1), lambda m: (m, 0)),
          pl.BlockSpec((H, H), lambda m: (0, 0)),
      ],
      out_specs=pl.BlockSpec((BM, H), lambda m: (m, 0)),
      out_shape=jax.ShapeDtypeStruct((NP, H), jnp.float32),
  )(s0, s1, hp, dis, linp)


def kernel(x, edge_index, emb_table, conv_W, lin_W):
  f32 = jnp.float32
  i32 = jnp.int32
  bag = x.shape[1]

  xp = jnp.pad(x.astype(i32), ((0, NP - N), (0, 0)))
  xflat2 = xp.reshape(NP * bag // CH, CH)
  bag_dst2 = (jnp.arange(NP * bag, dtype=i32) // bag).reshape(
      NP * bag // CH, CH)
  pad_e = EP - edge_index.shape[1]
  pad_i = jnp.arange(pad_e, dtype=i32)
  srcp = jnp.concatenate(
      [edge_index[0].astype(i32), pad_i % jnp.int32(N)])
  dstp = jnp.concatenate(
      [edge_index[1].astype(i32), jnp.int32(N) + pad_i % jnp.int32(NP - N)])
  src2 = srcp.reshape(EP // CH, CH)
  dst2 = dstp.reshape(EP // CH, CH)
  zeros_h = jnp.zeros((CH, H), f32)
  ones_fill = jnp.ones((CH, H), f32)
  ones_tab = jnp.ones((8, H), f32)
  linp = jnp.pad(lin_W.astype(f32), ((0, 0), (0, H - lin_W.shape[1])))

  deg = _sc_segsum(H, EP // NW, gather=False)(
      ones_tab, src2, dst2, zeros_h, ones_fill)
  h0 = _sc_segsum(H, NP * bag // NW, gather=True)(
      emb_table, xflat2, bag_dst2, zeros_h, zeros_h)

  hws, dis = _tc_first(h0[0], h0[1], deg[0, :, :1], deg[1, :, :1], conv_W[0])

  seg = _sc_segsum(H, EP // NW, gather=True)
  for i in range(1, 5):
    scat = seg(hws, src2, dst2, zeros_h, zeros_h)
    hws = _tc_mid(scat[0], scat[1], hws, dis, conv_W[i])
  scat = seg(hws, src2, dst2, zeros_h, zeros_h)
  probs = _tc_last(scat[0], scat[1], hws, dis, linp)
  return probs[:N, :lin_W.shape[1]]


# speedup vs baseline: 13.8386x; 1.3093x over previous
import functools

import jax
import jax.numpy as jnp
from jax import lax
from jax.experimental import pallas as pl
from jax.experimental.pallas import tpu as pltpu
from jax.experimental.pallas import tpu_sc as plsc

N = 10000
H = 128
NP = 10240
EP = 327680
NW = 32
TILES = 16
ROWS_PER_TILE = NP // TILES
CH = 128
J = 2
BM = 1024


def _sc_segsum(width, edges_per_tile, gather):
  n = edges_per_tile // CH
  blk = 20 if n % 20 == 0 else (10 if n % 10 == 0 else n)
  nchunks = n // blk
  mesh = plsc.VectorSubcoreMesh(core_axis_name="c", subcore_axis_name="s")

  def body(table, sd, z_op, fill_op, out, idxb, rows, acc, gsem, ssem):
    c = lax.axis_index("c")
    s = lax.axis_index("s")
    w = c * TILES + s
    r0 = s * ROWS_PER_TILE
    pltpu.sync_copy(z_op, rows.at[0])
    for k in range(ROWS_PER_TILE // CH):
      pltpu.sync_copy(rows.at[0], acc.at[pl.ds(r0 + k * CH, CH)])
    if not gather:
      pltpu.sync_copy(fill_op, rows.at[0])
      pltpu.sync_copy(fill_op, rows.at[1])
    plsc.subcore_barrier()

    tile_row0 = w * n

    pltpu.sync_copy(sd.at[pl.ds(tile_row0, blk)], idxb.at[0])
    if gather:
      pltpu.async_copy(table.at[idxb.at[0, 0, 0]], rows.at[0], gsem)

    @pl.loop(0, n // 2)
    def _(k):
      for b in range(2):
        t = 2 * k + b
        p, q = b, 1 - b
        cs = (t // blk) % 2
        if gather:
          pltpu.make_async_copy(table.at[idxb.at[cs, t % blk, 0]],
                                rows.at[p], gsem).wait()
        pltpu.async_copy(rows.at[p], acc.at[idxb.at[cs, t % blk, 1]],
                         ssem, add=True)

        @pl.when(t > 0)
        def _():
          tm = t - 1
          pltpu.make_async_copy(
              rows.at[q], acc.at[idxb.at[(tm // blk) % 2, tm % blk, 1]],
              ssem).wait()

        @pl.when((t % blk == blk - 1) & (t < n - 1))
        def _():
          nc = t // blk + 1
          pltpu.sync_copy(sd.at[pl.ds(tile_row0 + nc * blk, blk)],
                          idxb.at[nc % 2])

        @pl.when(t < n - 1)
        def _():
          tn = t + 1
          if gather:
            pltpu.async_copy(table.at[idxb.at[(tn // blk) % 2, tn % blk, 0]],
                             rows.at[q], gsem)

    lt = n - 1
    pltpu.make_async_copy(
        rows.at[lt % 2], acc.at[idxb.at[(lt // blk) % 2, lt % blk, 1]],
        ssem).wait()
    plsc.subcore_barrier()
    pltpu.sync_copy(acc.at[pl.ds(r0, ROWS_PER_TILE)],
                    out.at[c, pl.ds(r0, ROWS_PER_TILE)])

  return functools.partial(
      pl.kernel,
      out_type=jax.ShapeDtypeStruct((2, NP, width), jnp.float32),
      mesh=mesh,
      scratch_types=[
          pltpu.VMEM((2, blk, 2, CH), jnp.int32),
          pltpu.VMEM((2, CH, width), jnp.float32),
          pltpu.VMEM_SHARED((NP, width), jnp.float32),
          pltpu.SemaphoreType.DMA,
          pltpu.SemaphoreType.DMA,
      ],
  )(body)


def _tc_first(h0a, h0b, deg0, deg1, W0):
  def body(a_ref, b_ref, d0_ref, d1_ref, w_ref, hws_ref, dis_ref):
    dis = lax.rsqrt(d0_ref[...] + d1_ref[...] + 1.0)
    h = jnp.maximum((a_ref[...] + b_ref[...]) * (1.0 / 16.0), 0.0)
    hw = jnp.dot(h, w_ref[...], preferred_element_type=jnp.float32)
    hws_ref[...] = hw * dis
    dis_ref[...] = dis

  return pl.pallas_call(
      body,
      grid=(NP // BM,),
      in_specs=[
          pl.BlockSpec((BM, H), lambda m: (m, 0)),
          pl.BlockSpec((BM, H), lambda m: (m, 0)),
          pl.BlockSpec((BM, 1), lambda m: (m, 0)),
          pl.BlockSpec((BM, 1), lambda m: (m, 0)),
          pl.BlockSpec((H, H), lambda m: (0, 0)),
      ],
      out_specs=[
          pl.BlockSpec((BM, H), lambda m: (m, 0)),
          pl.BlockSpec((BM, 1), lambda m: (m, 0)),
      ],
      out_shape=[
          jax.ShapeDtypeStruct((NP, H), jnp.float32),
          jax.ShapeDtypeStruct((NP, 1), jnp.float32),
      ],
  )(h0a, h0b, deg0, deg1, W0)


def _tc_mid(s0, s1, hp, dis, W):
  def body(s0_ref, s1_ref, hp_ref, dis_ref, w_ref, out_ref):
    dis = dis_ref[...]
    t = jnp.maximum((s0_ref[...] + s1_ref[...] + hp_ref[...]) * dis, 0.0)
    out_ref[...] = jnp.dot(
        t, w_ref[...], preferred_element_type=jnp.float32) * dis

  return pl.pallas_call(
      body,
      grid=(NP // BM,),
      in_specs=[
          pl.BlockSpec((BM, H), lambda m: (m, 0)),
          pl.BlockSpec((BM, H), lambda m: (m, 0)),
          pl.BlockSpec((BM, H), lambda m: (m, 0)),
          pl.BlockSpec((BM, 1), lambda m: (m, 0)),
          pl.BlockSpec((H, H), lambda m: (0, 0)),
      ],
      out_specs=pl.BlockSpec((BM, H), lambda m: (m, 0)),
      out_shape=jax.ShapeDtypeStruct((NP, H), jnp.float32),
  )(s0, s1, hp, dis, W)


def _tc_last(s0, s1, hp, dis, linp):
  def body(s0_ref, s1_ref, hp_ref, dis_ref, w_ref, out_ref):
    t = jnp.maximum(
        (s0_ref[...] + s1_ref[...] + hp_ref[...]) * dis_ref[...], 0.0)
    z = jnp.dot(t, w_ref[...], preferred_element_type=jnp.float32)
    col = lax.broadcasted_iota(jnp.int32, (BM, H), 1)
    z = jnp.where(col < 7, z, -jnp.inf)
    m = jnp.max(z, axis=1, keepdims=True)
    e = jnp.exp(z - m)
    out_ref[...] = e / jnp.sum(e, axis=1, keepdims=True)

  return pl.pallas_call(
      body,
      grid=(NP // BM,),
      in_specs=[
          pl.BlockSpec((BM, H), lambda m: (m, 0)),
          pl.BlockSpec((BM, H), lambda m: (m, 0)),
          pl.BlockSpec((BM, H), lambda m: (m, 0)),
          pl.BlockSpec((BM, 1), lambda m: (m, 0)),
          pl.BlockSpec((H, H), lambda m: (0, 0)),
      ],
      out_specs=pl.BlockSpec((BM, H), lambda m: (m, 0)),
      out_shape=jax.ShapeDtypeStruct((NP, H), jnp.float32),
  )(s0, s1, hp, dis, linp)


def kernel(x, edge_index, emb_table, conv_W, lin_W):
  f32 = jnp.float32
  i32 = jnp.int32
  bag = x.shape[1]

  xp = jnp.pad(x.astype(i32), ((0, NP - N), (0, 0)))
  xflat2 = xp.reshape(NP * bag // CH, CH)
  bag_dst2 = (jnp.arange(NP * bag, dtype=i32) // bag).reshape(
      NP * bag // CH, CH)
  pad_e = EP - edge_index.shape[1]
  pad_i = jnp.arange(pad_e, dtype=i32)
  srcp = jnp.concatenate(
      [edge_index[0].astype(i32), pad_i % jnp.int32(N)])
  dstp = jnp.concatenate(
      [edge_index[1].astype(i32), jnp.int32(N) + pad_i % jnp.int32(NP - N)])
  sd_edges = jnp.stack(
      [srcp.reshape(EP // CH, CH), dstp.reshape(EP // CH, CH)], axis=1)
  sd_bag = jnp.stack([xflat2, bag_dst2], axis=1)
  zeros_h = jnp.zeros((CH, H), f32)
  ones_fill = jnp.ones((CH, H), f32)
  ones_tab = jnp.ones((8, H), f32)
  linp = jnp.pad(lin_W.astype(f32), ((0, 0), (0, H - lin_W.shape[1])))

  deg = _sc_segsum(H, EP // NW, gather=False)(
      ones_tab, sd_edges, zeros_h, ones_fill)
  h0 = _sc_segsum(H, NP * bag // NW, gather=True)(
      emb_table, sd_bag, zeros_h, zeros_h)

  hws, dis = _tc_first(h0[0], h0[1], deg[0, :, :1], deg[1, :, :1], conv_W[0])

  seg = _sc_segsum(H, EP // NW, gather=True)
  for i in range(1, 5):
    scat = seg(hws, sd_edges, zeros_h, zeros_h)
    hws = _tc_mid(scat[0], scat[1], hws, dis, conv_W[i])
  scat = seg(hws, sd_edges, zeros_h, zeros_h)
  probs = _tc_last(scat[0], scat[1], hws, dis, linp)
  return probs[:N, :lin_W.shape[1]]


# speedup vs baseline: 14.6626x; 1.0595x over previous
import functools

import jax
import jax.numpy as jnp
from jax import lax
from jax.experimental import pallas as pl
from jax.experimental.pallas import tpu as pltpu
from jax.experimental.pallas import tpu_sc as plsc

N = 10000
H = 128
NP = 10240
EP = 327680
NW = 32
TILES = 16
ROWS_PER_TILE = NP // TILES
CH = 128
J = 2
BM = 1024


def _sc_deg():
  n_steps = EP // NW // CH
  cols = NP // TILES
  mesh = plsc.VectorSubcoreMesh(core_axis_name="c", subcore_axis_name="s")

  def body(sd, out, idxb, hist, buf, red, acc):
    c = lax.axis_index("c")
    s = lax.axis_index("s")
    w = c * TILES + s
    pltpu.sync_copy(sd.at[pl.ds(w * n_steps, n_steps)], idxb)
    z16 = jnp.zeros((16,), jnp.float32)
    o16 = jnp.ones((16,), jnp.float32)

    @pl.loop(0, NP // 16)
    def _(g):
      hist[pl.ds(g * 16, 16)] = z16

    @pl.loop(0, n_steps)
    def _(t):
      for g in range(CH // 16):
        idx = idxb[t, 1, pl.ds(g * 16, 16)]
        plsc.addupdate_scatter(hist, [idx], o16)

    pltpu.sync_copy(hist, acc.at[s])
    plsc.subcore_barrier()
    pltpu.sync_copy(acc.at[pl.ds(0, TILES), pl.ds(s * cols, cols)], buf)

    @pl.loop(0, cols // 16)
    def _(j):
      v = buf[0, pl.ds(j * 16, 16)]
      for r in range(1, TILES):
        v = v + buf[r, pl.ds(j * 16, 16)]
      red[pl.ds(j * 16, 16)] = v

    pltpu.sync_copy(red, out.at[c, pl.ds(s * cols, cols)])

  return functools.partial(
      pl.kernel,
      out_type=jax.ShapeDtypeStruct((2, NP), jnp.float32),
      mesh=mesh,
      scratch_types=[
          pltpu.VMEM((n_steps, 2, CH), jnp.int32),
          pltpu.VMEM((NP,), jnp.float32),
          pltpu.VMEM((TILES, cols), jnp.float32),
          pltpu.VMEM((cols,), jnp.float32),
          pltpu.VMEM_SHARED((TILES, NP), jnp.float32),
      ],
      compiler_params=pltpu.CompilerParams(needs_layout_passes=False),
  )(body)


def _sc_segsum(width, edges_per_tile):
  n = edges_per_tile // CH
  blk = 20 if n % 20 == 0 else (10 if n % 10 == 0 else n)
  nchunks = n // blk
  mesh = plsc.VectorSubcoreMesh(core_axis_name="c", subcore_axis_name="s")

  def body(table, sd, z_op, out, idxb, rows, acc, gsem, ssem):
    c = lax.axis_index("c")
    s = lax.axis_index("s")
    w = c * TILES + s
    r0 = s * ROWS_PER_TILE
    pltpu.sync_copy(z_op, rows.at[0])
    for k in range(ROWS_PER_TILE // CH):
      pltpu.sync_copy(rows.at[0], acc.at[pl.ds(r0 + k * CH, CH)])
    plsc.subcore_barrier()

    tile_row0 = w * n

    pltpu.sync_copy(sd.at[pl.ds(tile_row0, blk)], idxb.at[0])
    pltpu.async_copy(table.at[idxb.at[0, 0, 0]], rows.at[0], gsem)

    @pl.loop(0, n // 2)
    def _(k):
      for b in range(2):
        t = 2 * k + b
        p, q = b, 1 - b
        cs = (t // blk) % 2
        pltpu.make_async_copy(table.at[idxb.at[cs, t % blk, 0]],
                              rows.at[p], gsem).wait()
        pltpu.async_copy(rows.at[p], acc.at[idxb.at[cs, t % blk, 1]],
                         ssem, add=True)

        @pl.when(t > 0)
        def _():
          tm = t - 1
          pltpu.make_async_copy(
              rows.at[q], acc.at[idxb.at[(tm // blk) % 2, tm % blk, 1]],
              ssem).wait()

        @pl.when((t % blk == blk - 1) & (t < n - 1))
        def _():
          nc = t // blk + 1
          pltpu.sync_copy(sd.at[pl.ds(tile_row0 + nc * blk, blk)],
                          idxb.at[nc % 2])

        @pl.when(t < n - 1)
        def _():
          tn = t + 1
          pltpu.async_copy(table.at[idxb.at[(tn // blk) % 2, tn % blk, 0]],
                           rows.at[q], gsem)

    lt = n - 1
    pltpu.make_async_copy(
        rows.at[lt % 2], acc.at[idxb.at[(lt // blk) % 2, lt % blk, 1]],
        ssem).wait()
    plsc.subcore_barrier()
    pltpu.sync_copy(acc.at[pl.ds(r0, ROWS_PER_TILE)],
                    out.at[c, pl.ds(r0, ROWS_PER_TILE)])

  return functools.partial(
      pl.kernel,
      out_type=jax.ShapeDtypeStruct((2, NP, width), jnp.float32),
      mesh=mesh,
      scratch_types=[
          pltpu.VMEM((2, blk, 2, CH), jnp.int32),
          pltpu.VMEM((2, CH, width), jnp.float32),
          pltpu.VMEM_SHARED((NP, width), jnp.float32),
          pltpu.SemaphoreType.DMA,
          pltpu.SemaphoreType.DMA,
      ],
  )(body)


def _tc_first(h0a, h0b, deg0, deg1, W0):
  def body(a_ref, b_ref, d0_ref, d1_ref, w_ref, hws_ref, dis_ref):
    dis = lax.rsqrt(d0_ref[...] + d1_ref[...] + 1.0)
    h = jnp.maximum((a_ref[...] + b_ref[...]) * (1.0 / 16.0), 0.0)
    hw = jnp.dot(h, w_ref[...], preferred_element_type=jnp.float32)
    hws_ref[...] = hw * dis
    dis_ref[...] = dis

  return pl.pallas_call(
      body,
      grid=(NP // BM,),
      in_specs=[
          pl.BlockSpec((BM, H), lambda m: (m, 0)),
          pl.BlockSpec((BM, H), lambda m: (m, 0)),
          pl.BlockSpec((BM, 1), lambda m: (m, 0)),
          pl.BlockSpec((BM, 1), lambda m: (m, 0)),
          pl.BlockSpec((H, H), lambda m: (0, 0)),
      ],
      out_specs=[
          pl.BlockSpec((BM, H), lambda m: (m, 0)),
          pl.BlockSpec((BM, 1), lambda m: (m, 0)),
      ],
      out_shape=[
          jax.ShapeDtypeStruct((NP, H), jnp.float32),
          jax.ShapeDtypeStruct((NP, 1), jnp.float32),
      ],
  )(h0a, h0b, deg0, deg1, W0)


def _tc_mid(s0, s1, hp, dis, W):
  def body(s0_ref, s1_ref, hp_ref, dis_ref, w_ref, out_ref):
    dis = dis_ref[...]
    t = jnp.maximum((s0_ref[...] + s1_ref[...] + hp_ref[...]) * dis, 0.0)
    out_ref[...] = jnp.dot(
        t, w_ref[...], preferred_element_type=jnp.float32) * dis

  return pl.pallas_call(
      body,
      grid=(NP // BM,),
      in_specs=[
          pl.BlockSpec((BM, H), lambda m: (m, 0)),
          pl.BlockSpec((BM, H), lambda m: (m, 0)),
          pl.BlockSpec((BM, H), lambda m: (m, 0)),
          pl.BlockSpec((BM, 1), lambda m: (m, 0)),
          pl.BlockSpec((H, H), lambda m: (0, 0)),
      ],
      out_specs=pl.BlockSpec((BM, H), lambda m: (m, 0)),
      out_shape=jax.ShapeDtypeStruct((NP, H), jnp.float32),
  )(s0, s1, hp, dis, W)


def _tc_last(s0, s1, hp, dis, linp):
  def body(s0_ref, s1_ref, hp_ref, dis_ref, w_ref, out_ref):
    t = jnp.maximum(
        (s0_ref[...] + s1_ref[...] + hp_ref[...]) * dis_ref[...], 0.0)
    z = jnp.dot(t, w_ref[...], preferred_element_type=jnp.float32)
    col = lax.broadcasted_iota(jnp.int32, (BM, H), 1)
    z = jnp.where(col < 7, z, -jnp.inf)
    m = jnp.max(z, axis=1, keepdims=True)
    e = jnp.exp(z - m)
    out_ref[...] = e / jnp.sum(e, axis=1, keepdims=True)

  return pl.pallas_call(
      body,
      grid=(NP // BM,),
      in_specs=[
          pl.BlockSpec((BM, H), lambda m: (m, 0)),
          pl.BlockSpec((BM, H), lambda m: (m, 0)),
          pl.BlockSpec((BM, H), lambda m: (m, 0)),
          pl.BlockSpec((BM, 1), lambda m: (m, 0)),
          pl.BlockSpec((H, H), lambda m: (0, 0)),
      ],
      out_specs=pl.BlockSpec((BM, H), lambda m: (m, 0)),
      out_shape=jax.ShapeDtypeStruct((NP, H), jnp.float32),
  )(s0, s1, hp, dis, linp)


def kernel(x, edge_index, emb_table, conv_W, lin_W):
  f32 = jnp.float32
  i32 = jnp.int32
  bag = x.shape[1]

  xp = jnp.pad(x.astype(i32), ((0, NP - N), (0, 0)))
  xflat2 = xp.reshape(NP * bag // CH, CH)
  bag_dst2 = (jnp.arange(NP * bag, dtype=i32) // bag).reshape(
      NP * bag // CH, CH)
  pad_e = EP - edge_index.shape[1]
  pad_i = jnp.arange(pad_e, dtype=i32)
  srcp = jnp.concatenate(
      [edge_index[0].astype(i32), pad_i % jnp.int32(N)])
  dstp = jnp.concatenate(
      [edge_index[1].astype(i32), jnp.int32(N) + pad_i % jnp.int32(NP - N)])
  sd_edges = jnp.stack(
      [srcp.reshape(EP // CH, CH), dstp.reshape(EP // CH, CH)], axis=1)
  sd_bag = jnp.stack([xflat2, bag_dst2], axis=1)
  zeros_h = jnp.zeros((CH, H), f32)
  linp = jnp.pad(lin_W.astype(f32), ((0, 0), (0, H - lin_W.shape[1])))

  deg = _sc_deg()(sd_edges)
  h0 = _sc_segsum(H, NP * bag // NW)(emb_table, sd_bag, zeros_h)

  hws, dis = _tc_first(h0[0], h0[1], deg[0].reshape(NP, 1),
                       deg[1].reshape(NP, 1), conv_W[0])

  seg = _sc_segsum(H, EP // NW)
  for i in range(1, 5):
    scat = seg(hws, sd_edges, zeros_h)
    hws = _tc_mid(scat[0], scat[1], hws, dis, conv_W[i])
  scat = seg(hws, sd_edges, zeros_h)
  probs = _tc_last(scat[0], scat[1], hws, dis, linp)
  return probs[:N, :lin_W.shape[1]]


# speedup vs baseline: 17.9930x; 1.2271x over previous
import functools

import jax
import jax.numpy as jnp
from jax import lax
from jax.experimental import pallas as pl
from jax.experimental.pallas import tpu as pltpu
from jax.experimental.pallas import tpu_sc as plsc

N = 10000
H = 128
NP = 10240
EP = 327680
NW = 32
TILES = 16
ROWS_PER_TILE = NP // TILES
CH = 128
J = 2
BM = 1024


def _sc_deg():
  n_steps = EP // NW // CH
  cols = NP // TILES
  mesh = plsc.VectorSubcoreMesh(core_axis_name="c", subcore_axis_name="s")

  def body(sd, out, idxb, hist, buf, red, acc):
    c = lax.axis_index("c")
    s = lax.axis_index("s")
    w = c * TILES + s
    pltpu.sync_copy(sd.at[pl.ds(w * n_steps, n_steps)], idxb)
    z16 = jnp.zeros((16,), jnp.float32)
    o16 = jnp.ones((16,), jnp.float32)

    @pl.loop(0, NP // 16)
    def _(g):
      hist[pl.ds(g * 16, 16)] = z16

    @pl.loop(0, n_steps)
    def _(t):
      for g in range(CH // 16):
        idx = idxb[t, 1, pl.ds(g * 16, 16)]
        plsc.addupdate_scatter(hist, [idx], o16)

    pltpu.sync_copy(hist, acc.at[s])
    plsc.subcore_barrier()
    pltpu.sync_copy(acc.at[pl.ds(0, TILES), pl.ds(s * cols, cols)], buf)

    @pl.loop(0, cols // 16)
    def _(j):
      v = buf[0, pl.ds(j * 16, 16)]
      for r in range(1, TILES):
        v = v + buf[r, pl.ds(j * 16, 16)]
      red[pl.ds(j * 16, 16)] = v

    pltpu.sync_copy(red, out.at[c, pl.ds(s * cols, cols)])

  return functools.partial(
      pl.kernel,
      out_type=jax.ShapeDtypeStruct((2, NP), jnp.float32),
      mesh=mesh,
      scratch_types=[
          pltpu.VMEM((n_steps, 2, CH), jnp.int32),
          pltpu.VMEM((NP,), jnp.float32),
          pltpu.VMEM((TILES, cols), jnp.float32),
          pltpu.VMEM((cols,), jnp.float32),
          pltpu.VMEM_SHARED((TILES, NP), jnp.float32),
      ],
      compiler_params=pltpu.CompilerParams(needs_layout_passes=False),
  )(body)


def _sc_emb():
  n = NP * 16 // NW // CH
  blk = 20
  npt = NP // NW
  tpr = 1536 // TILES
  mesh = plsc.VectorSubcoreMesh(core_axis_name="c", subcore_axis_name="s")

  def body(tabp, sd, z_op, out, idxb, rows, vacc, sptab, gsem, ssem):
    c = lax.axis_index("c")
    s = lax.axis_index("s")
    w = c * TILES + s
    tr0 = s * tpr
    a0 = s * npt
    pltpu.sync_copy(tabp.at[pl.ds(tr0, tpr)], rows.at[0, pl.ds(0, tpr)])
    pltpu.sync_copy(rows.at[0, pl.ds(0, tpr)], sptab.at[pl.ds(tr0, tpr)])
    pltpu.sync_copy(z_op, vacc.at[pl.ds(a0, CH)])
    pltpu.sync_copy(z_op, vacc.at[pl.ds(a0 + CH, CH)])
    pltpu.sync_copy(z_op.at[pl.ds(0, npt - 2 * CH)],
                    vacc.at[pl.ds(a0 + 2 * CH, npt - 2 * CH)])
    plsc.subcore_barrier()

    tile_row0 = w * n
    pltpu.sync_copy(sd.at[pl.ds(tile_row0, blk)], idxb.at[0])
    pltpu.async_copy(sptab.at[idxb.at[0, 0, 0]], rows.at[0], gsem)

    @pl.loop(0, n // 2)
    def _(k):
      for b in range(2):
        t = 2 * k + b
        p, q = b, 1 - b
        cs = (t // blk) % 2
        pltpu.make_async_copy(sptab.at[idxb.at[cs, t % blk, 0]],
                              rows.at[p], gsem).wait()
        pltpu.async_copy(rows.at[p], vacc.at[idxb.at[cs, t % blk, 1]],
                         ssem, add=True)

        @pl.when(t > 0)
        def _():
          tm = t - 1
          pltpu.make_async_copy(
              rows.at[q], vacc.at[idxb.at[(tm // blk) % 2, tm % blk, 1]],
              ssem).wait()

        @pl.when((t % blk == blk - 1) & (t < n - 1))
        def _():
          nc = t // blk + 1
          pltpu.sync_copy(sd.at[pl.ds(tile_row0 + nc * blk, blk)],
                          idxb.at[nc % 2])

        @pl.when(t < n - 1)
        def _():
          tn = t + 1
          pltpu.async_copy(sptab.at[idxb.at[(tn // blk) % 2, tn % blk, 0]],
                           rows.at[q], gsem)

    lt = n - 1
    pltpu.make_async_copy(
        rows.at[lt % 2], vacc.at[idxb.at[(lt // blk) % 2, lt % blk, 1]],
        ssem).wait()
    pltpu.sync_copy(vacc.at[pl.ds(a0, npt)], out.at[pl.ds(w * npt, npt)])

  return functools.partial(
      pl.kernel,
      out_type=jax.ShapeDtypeStruct((NP, H), jnp.float32),
      mesh=mesh,
      scratch_types=[
          pltpu.VMEM((2, blk, 2, CH), jnp.int32),
          pltpu.VMEM((2, CH, H), jnp.float32),
          pltpu.VMEM_SHARED((NP // 2, H), jnp.float32),
          pltpu.VMEM_SHARED((1536, H), jnp.float32),
          pltpu.SemaphoreType.DMA,
          pltpu.SemaphoreType.DMA,
      ],
  )(body)


def _sc_segsum(width, edges_per_tile):
  n = edges_per_tile // CH
  blk = 20 if n % 20 == 0 else (10 if n % 10 == 0 else n)
  nchunks = n // blk
  mesh = plsc.VectorSubcoreMesh(core_axis_name="c", subcore_axis_name="s")

  def body(table, sd, z_op, out, idxb, rows, acc, gsem, ssem):
    c = lax.axis_index("c")
    s = lax.axis_index("s")
    w = c * TILES + s
    r0 = s * ROWS_PER_TILE
    pltpu.sync_copy(z_op, rows.at[0])
    for k in range(ROWS_PER_TILE // CH):
      pltpu.sync_copy(rows.at[0], acc.at[pl.ds(r0 + k * CH, CH)])
    plsc.subcore_barrier()

    tile_row0 = w * n

    pltpu.sync_copy(sd.at[pl.ds(tile_row0, blk)], idxb.at[0])
    pltpu.async_copy(table.at[idxb.at[0, 0, 0]], rows.at[0], gsem)

    @pl.loop(0, n // 2)
    def _(k):
      for b in range(2):
        t = 2 * k + b
        p, q = b, 1 - b
        cs = (t // blk) % 2
        pltpu.make_async_copy(table.at[idxb.at[cs, t % blk, 0]],
                              rows.at[p], gsem).wait()
        pltpu.async_copy(rows.at[p], acc.at[idxb.at[cs, t % blk, 1]],
                         ssem, add=True)

        @pl.when(t > 0)
        def _():
          tm = t - 1
          pltpu.make_async_copy(
              rows.at[q], acc.at[idxb.at[(tm // blk) % 2, tm % blk, 1]],
              ssem).wait()

        @pl.when((t % blk == blk - 1) & (t < n - 1))
        def _():
          nc = t // blk + 1
          pltpu.sync_copy(sd.at[pl.ds(tile_row0 + nc * blk, blk)],
                          idxb.at[nc % 2])

        @pl.when(t < n - 1)
        def _():
          tn = t + 1
          pltpu.async_copy(table.at[idxb.at[(tn // blk) % 2, tn % blk, 0]],
                           rows.at[q], gsem)

    lt = n - 1
    pltpu.make_async_copy(
        rows.at[lt % 2], acc.at[idxb.at[(lt // blk) % 2, lt % blk, 1]],
        ssem).wait()
    plsc.subcore_barrier()
    pltpu.sync_copy(acc.at[pl.ds(r0, ROWS_PER_TILE)],
                    out.at[c, pl.ds(r0, ROWS_PER_TILE)])

  return functools.partial(
      pl.kernel,
      out_type=jax.ShapeDtypeStruct((2, NP, width), jnp.float32),
      mesh=mesh,
      scratch_types=[
          pltpu.VMEM((2, blk, 2, CH), jnp.int32),
          pltpu.VMEM((2, CH, width), jnp.float32),
          pltpu.VMEM_SHARED((NP, width), jnp.float32),
          pltpu.SemaphoreType.DMA,
          pltpu.SemaphoreType.DMA,
      ],
  )(body)


def _tc_first(h0, deg0, deg1, W0):
  def body(a_ref, d0_ref, d1_ref, w_ref, hws_ref, dis_ref):
    dis = lax.rsqrt(d0_ref[...] + d1_ref[...] + 1.0)
    h = jnp.maximum(a_ref[...] * (1.0 / 16.0), 0.0)
    hw = jnp.dot(h, w_ref[...], preferred_element_type=jnp.float32)
    hws_ref[...] = hw * dis
    dis_ref[...] = dis

  return pl.pallas_call(
      body,
      grid=(NP // BM,),
      in_specs=[
          pl.BlockSpec((BM, H), lambda m: (m, 0)),
          pl.BlockSpec((BM, 1), lambda m: (m, 0)),
          pl.BlockSpec((BM, 1), lambda m: (m, 0)),
          pl.BlockSpec((H, H), lambda m: (0, 0)),
      ],
      out_specs=[
          pl.BlockSpec((BM, H), lambda m: (m, 0)),
          pl.BlockSpec((BM, 1), lambda m: (m, 0)),
      ],
      out_shape=[
          jax.ShapeDtypeStruct((NP, H), jnp.float32),
          jax.ShapeDtypeStruct((NP, 1), jnp.float32),
      ],
  )(h0, deg0, deg1, W0)


def _tc_mid(s0, s1, hp, dis, W):
  def body(s0_ref, s1_ref, hp_ref, dis_ref, w_ref, out_ref):
    dis = dis_ref[...]
    t = jnp.maximum((s0_ref[...] + s1_ref[...] + hp_ref[...]) * dis, 0.0)
    out_ref[...] = jnp.dot(
        t, w_ref[...], preferred_element_type=jnp.float32) * dis

  return pl.pallas_call(
      body,
      grid=(NP // BM,),
      in_specs=[
          pl.BlockSpec((BM, H), lambda m: (m, 0)),
          pl.BlockSpec((BM, H), lambda m: (m, 0)),
          pl.BlockSpec((BM, H), lambda m: (m, 0)),
          pl.BlockSpec((BM, 1), lambda m: (m, 0)),
          pl.BlockSpec((H, H), lambda m: (0, 0)),
      ],
      out_specs=pl.BlockSpec((BM, H), lambda m: (m, 0)),
      out_shape=jax.ShapeDtypeStruct((NP, H), jnp.float32),
  )(s0, s1, hp, dis, W)


def _tc_last(s0, s1, hp, dis, linp):
  def body(s0_ref, s1_ref, hp_ref, dis_ref, w_ref, out_ref):
    t = jnp.maximum(
        (s0_ref[...] + s1_ref[...] + hp_ref[...]) * dis_ref[...], 0.0)
    z = jnp.dot(t, w_ref[...], preferred_element_type=jnp.float32)
    col = lax.broadcasted_iota(jnp.int32, (BM, H), 1)
    z = jnp.where(col < 7, z, -jnp.inf)
    m = jnp.max(z, axis=1, keepdims=True)
    e = jnp.exp(z - m)
    out_ref[...] = e / jnp.sum(e, axis=1, keepdims=True)

  return pl.pallas_call(
      body,
      grid=(NP // BM,),
      in_specs=[
          pl.BlockSpec((BM, H), lambda m: (m, 0)),
          pl.BlockSpec((BM, H), lambda m: (m, 0)),
          pl.BlockSpec((BM, H), lambda m: (m, 0)),
          pl.BlockSpec((BM, 1), lambda m: (m, 0)),
          pl.BlockSpec((H, H), lambda m: (0, 0)),
      ],
      out_specs=pl.BlockSpec((BM, H), lambda m: (m, 0)),
      out_shape=jax.ShapeDtypeStruct((NP, H), jnp.float32),
  )(s0, s1, hp, dis, linp)


def kernel(x, edge_index, emb_table, conv_W, lin_W):
  f32 = jnp.float32
  i32 = jnp.int32
  bag = x.shape[1]

  xp = jnp.pad(x.astype(i32), ((0, NP - N), (0, 0)))
  xflat2 = xp.reshape(NP * bag // CH, CH)
  bag_dst2 = ((jnp.arange(NP * bag, dtype=i32) // bag) % (NP // 2)).reshape(
      NP * bag // CH, CH)
  emb_pad = jnp.pad(emb_table.astype(f32), ((0, 1536 - emb_table.shape[0]),
                                            (0, 0)))
  pad_e = EP - edge_index.shape[1]
  pad_i = jnp.arange(pad_e, dtype=i32)
  srcp = jnp.concatenate(
      [edge_index[0].astype(i32), pad_i % jnp.int32(N)])
  dstp = jnp.concatenate(
      [edge_index[1].astype(i32), jnp.int32(N) + pad_i % jnp.int32(NP - N)])
  sd_edges = jnp.stack(
      [srcp.reshape(EP // CH, CH), dstp.reshape(EP // CH, CH)], axis=1)
  sd_bag = jnp.stack([xflat2, bag_dst2], axis=1)
  zeros_h = jnp.zeros((CH, H), f32)
  linp = jnp.pad(lin_W.astype(f32), ((0, 0), (0, H - lin_W.shape[1])))

  deg = _sc_deg()(sd_edges)
  h0 = _sc_emb()(emb_pad, sd_bag, zeros_h)

  hws, dis = _tc_first(h0, deg[0].reshape(NP, 1),
                       deg[1].reshape(NP, 1), conv_W[0])

  seg = _sc_segsum(H, EP // NW)
  for i in range(1, 5):
    scat = seg(hws, sd_edges, zeros_h)
    hws = _tc_mid(scat[0], scat[1], hws, dis, conv_W[i])
  scat = seg(hws, sd_edges, zeros_h)
  probs = _tc_last(scat[0], scat[1], hws, dis, linp)
  return probs[:N, :lin_W.shape[1]]
